# Initial kernel scaffold; baseline (speedup 1.0000x reference)
import jax
import jax.numpy as jnp
from jax.experimental import pallas as pl


def kernel(query, values, W1_w, W1_b, V_w, V_b):
    raise NotImplementedError("write your pallas kernel here")


# speedup vs baseline: 22.1239x; 22.1239x over previous
import jax
import jax.numpy as jnp
from jax.experimental import pallas as pl


def _scores_body(q_ref, w1_ref, b1_ref, vw_ref, vb_ref, out_ref, *, rows):
    for i in range(rows):
        h = jnp.tanh(jnp.dot(q_ref[i], w1_ref[...],
                             preferred_element_type=jnp.float32) + b1_ref[0])
        out_ref[i, :] = jnp.sum(h * vw_ref[0], axis=1) + vb_ref[0, 0]


def _nms_body(raw_ref, cv_ref, ci_ref, slot_ref, *, R, T, C, radius, nstep,
              kslot):
    raw = raw_ref[...]
    neg = -jnp.inf
    pos = jax.lax.broadcasted_iota(jnp.int32, (1, T), 1)
    alive = jnp.ones((R, T), dtype=jnp.bool_)
    chosen = jnp.zeros((R, T), dtype=jnp.bool_)
    for _ in range(nstep):
        cand = jnp.where(alive & ~chosen, raw, neg)
        m = jnp.max(cand, axis=1, keepdims=True)
        active = m > neg
        is_max = (cand == m) & active
        idx = jnp.min(jnp.where(is_max, pos, T), axis=1, keepdims=True)
        sel = pos == idx
        nb = (pos >= idx - radius) & (pos <= idx + radius) & active
        alive = (alive & ~nb) | sel
        chosen = chosen | sel
    probs = jnp.where(chosen, 1.0 / (1.0 + jnp.exp(-raw)), 0.0)
    tril = (jax.lax.broadcasted_iota(jnp.int32, (T, T), 0)
            <= jax.lax.broadcasted_iota(jnp.int32, (T, T), 1)
            ).astype(jnp.float32)
    cumc = jnp.dot(chosen.astype(jnp.float32), tril,
                   preferred_element_type=jnp.float32).astype(jnp.int32)
    slot = jnp.where(chosen, cumc - 1, -1)
    slot_ref[...] = slot.astype(jnp.float32)
    k_iota = jax.lax.broadcasted_iota(jnp.int32, (1, 1, kslot), 2)
    Mf = (slot[:, :, None] == k_iota).astype(jnp.float32)
    cv_ref[...] = jnp.sum(probs[:, :, None] * Mf, axis=1)
    rowc = jax.lax.broadcasted_iota(jnp.int32, (R, 1), 0) % C
    gidx = (rowc * T + pos).astype(jnp.float32)
    ci_ref[...] = jnp.sum(gidx[:, :, None] * Mf, axis=1)


def _topp_body(cv_ref, ci_ref, cm_ref, ssum_ref, *, CK, p):
    v = cv_ref[0]
    ix = ci_ref[0]
    eye = (jax.lax.broadcasted_iota(jnp.int32, (CK, CK), 0)
           == jax.lax.broadcasted_iota(jnp.int32, (CK, CK), 1)
           ).astype(jnp.float32)
    v_col = jax.lax.dot_general(eye, v, (((1,), (1,)), ((), ())),
                                preferred_element_type=jnp.float32)
    ix_col = jax.lax.dot_general(eye, ix, (((1,), (1,)), ((), ())),
                                 preferred_element_type=jnp.float32)
    before = ((v_col > v) | ((v_col == v) & (ix_col < ix))
              ).astype(jnp.float32)
    G = jnp.dot(v, before, preferred_element_type=jnp.float32)
    rank = jnp.sum(before, axis=0, keepdims=True)
    S = jnp.sum(v)
    cum = (G + v) / (S + 1e-8)
    keep = ((cum <= p) | (rank < 3.0)) & (v > 0)
    cm = v * keep.astype(jnp.float32)
    cm_ref[0] = cm
    ssum_ref[0] = jnp.full(ssum_ref.shape[1:], jnp.sum(cm), jnp.float32)


def _scatter_body(cm_ref, slot_ref, ssum_ref, masked_ref, attn_ref, *, kslot,
                  n_total):
    slot = slot_ref[...].astype(jnp.int32)
    k_iota = jax.lax.broadcasted_iota(jnp.int32, (1, 1, kslot), 2)
    Mf = (slot[:, :, None] == k_iota).astype(jnp.float32)
    masked = jnp.sum(cm_ref[...][:, None, :] * Mf, axis=2)
    masked_ref[...] = masked
    ssum = ssum_ref[...]
    attn_ref[...] = jnp.where(ssum <= 0.0, 1.0 / n_total,
                              masked / (ssum + 1e-8))


def _context_body(attn_ref, vals_ref, out_ref):
    out_ref[0] = jax.lax.dot_general(
        attn_ref[0], vals_ref[0], (((1,), (0,)), ((), ())),
        preferred_element_type=jnp.float32)


def kernel(query, values, W1_w, W1_b, V_w, V_b):
    B, C, T, D_in = query.shape
    D_hid = W1_w.shape[1]
    R = B * C
    radius = int(round(0.05 * T))
    nstep = -(-T // (radius + 1))
    kslot = 32
    assert nstep <= kslot
    CK = C * kslot
    f32 = jnp.float32

    q = query.reshape(R, T, D_in)
    rows_blk = 8
    import functools

    raw = pl.pallas_call(
        functools.partial(_scores_body, rows=rows_blk),
        grid=(R // rows_blk,),
        in_specs=[
            pl.BlockSpec((rows_blk, T, D_in), lambda r: (r, 0, 0)),
            pl.BlockSpec((D_in, D_hid), lambda r: (0, 0)),
            pl.BlockSpec((1, D_hid), lambda r: (0, 0)),
            pl.BlockSpec((1, D_hid), lambda r: (0, 0)),
            pl.BlockSpec((1, 1), lambda r: (0, 0)),
        ],
        out_specs=pl.BlockSpec((rows_blk, T), lambda r: (r, 0)),
        out_shape=jax.ShapeDtypeStruct((R, T), f32),
    )(q, W1_w, W1_b.reshape(1, D_hid), V_w.reshape(1, D_hid),
      V_b.reshape(1, 1))

    cv, ci, slot = pl.pallas_call(
        functools.partial(_nms_body, R=R, T=T, C=C, radius=radius,
                          nstep=nstep, kslot=kslot),
        out_shape=(jax.ShapeDtypeStruct((R, kslot), f32),
                   jax.ShapeDtypeStruct((R, kslot), f32),
                   jax.ShapeDtypeStruct((R, T), f32)),
    )(raw)

    cm, ssum = pl.pallas_call(
        functools.partial(_topp_body, CK=CK, p=0.7),
        grid=(B,),
        in_specs=[
            pl.BlockSpec((1, 1, CK), lambda b: (b, 0, 0)),
            pl.BlockSpec((1, 1, CK), lambda b: (b, 0, 0)),
        ],
        out_specs=(pl.BlockSpec((1, 1, CK), lambda b: (b, 0, 0)),
                   pl.BlockSpec((1, 1, 128), lambda b: (b, 0, 0))),
        out_shape=(jax.ShapeDtypeStruct((B, 1, CK), f32),
                   jax.ShapeDtypeStruct((B, 1, 128), f32)),
    )(cv.reshape(B, 1, CK), ci.reshape(B, 1, CK))

    ssum40 = jnp.repeat(ssum[:, 0, :1], C, axis=0)
    masked, attn = pl.pallas_call(
        functools.partial(_scatter_body, kslot=kslot, n_total=C * T),
        out_shape=(jax.ShapeDtypeStruct((R, T), f32),
                   jax.ShapeDtypeStruct((R, T), f32)),
    )(cm.reshape(R, kslot), slot, ssum40)

    context = pl.pallas_call(
        _context_body,
        grid=(B,),
        in_specs=[
            pl.BlockSpec((1, 1, C * T), lambda b: (b, 0, 0)),
            pl.BlockSpec((1, C * T, D_in), lambda b: (b, 0, 0)),
        ],
        out_specs=pl.BlockSpec((1, 1, D_in), lambda b: (b, 0, 0)),
        out_shape=jax.ShapeDtypeStruct((B, 1, D_in), f32),
    )(attn.reshape(B, 1, C * T), values.reshape(B, C * T, D_in))

    return (context.reshape(B, D_in), attn.reshape(B, C, T, 1),
            masked.reshape(B, C, T, 1))
